# Initial kernel scaffold; baseline (speedup 1.0000x reference)
import jax
import jax.numpy as jnp
from jax.experimental import pallas as pl


def kernel(x, edge_index, Wl0, bl0, Wr0, br0, g0, beta0, Wl1, bl1, Wr1, br1, g1, beta1):
    raise NotImplementedError("write your pallas kernel here")


# speedup vs baseline: 6.0393x; 6.0393x over previous
import functools

import jax
import jax.numpy as jnp
from jax import lax
from jax.experimental import pallas as pl
from jax.experimental.pallas import tpu as pltpu
from jax.experimental.pallas import tpu_sc as plsc

_N = 10000
_D = 128
_E = 320000

_NC = 2
_NS = 16
_NW = _NC * _NS

_K = 128
_NCH = 79
_EPAD = _NW * _NCH * _K
_NPAD = 10240
_RPT = _NPAD // _NS
_RCH = _RPT // _K


def _make_sc_agg(width):
  mesh = plsc.VectorSubcoreMesh(core_axis_name="c", subcore_axis_name="s")
  out_type = jax.ShapeDtypeStruct((_NC * _NPAD, width), jnp.float32)
  scratch = [
      pltpu.VMEM((_K,), jnp.int32),
      pltpu.VMEM((_K,), jnp.int32),
      pltpu.VMEM((_K, width), jnp.float32),
      pltpu.VMEM_SHARED((_NPAD, width), jnp.float32),
      pltpu.SemaphoreType.DMA,
  ]

  def body(x_hbm, src_hbm, dst_hbm, z_hbm, out_sum, src_v, dst_v, msgs,
           sum_sh, sem):
    c = lax.axis_index("c")
    s = lax.axis_index("s")
    wid = s * _NC + c
    ebase = wid * _NCH
    base = s * _RPT

    pltpu.sync_copy(z_hbm, sum_sh.at[pl.ds(base, _RPT)])
    plsc.subcore_barrier()

    def chunk(j, carry):
      pltpu.sync_copy(src_hbm.at[ebase + j], src_v)
      pltpu.sync_copy(dst_hbm.at[ebase + j], dst_v)
      pltpu.async_copy(x_hbm.at[src_v], msgs, sem).wait()
      pltpu.sync_copy(msgs, sum_sh.at[dst_v], add=True)
      return carry

    lax.fori_loop(0, _NCH, chunk, 0)
    plsc.subcore_barrier()

    obase = c * _NPAD + base
    pltpu.sync_copy(sum_sh.at[pl.ds(base, _RPT)], out_sum.at[pl.ds(obase, _RPT)])

  return pl.kernel(
      body, out_type=out_type, mesh=mesh, scratch_types=scratch,
      compiler_params=pltpu.CompilerParams(use_tc_tiling_on_sc=False))


_DW = _D + 16
_sc_agg_deg = _make_sc_agg(_DW)
_sc_agg_nodeg = _make_sc_agg(_D)


def _make_dense(last):
  R = 1024

  def body(s0, s1, d0, d1, x_r, wl, bl, wr, br, g, beta, o_r):
    ssum = s0[...] + s1[...]
    deg = d0[...] + d1[...]
    degc = jnp.maximum(deg[:, 0:1], 1.0)
    mean = ssum / degc
    xv = x_r[...]
    dn = (((1,), (1,)), ((), ()))
    out = (lax.dot_general(mean, wl[...], dn, preferred_element_type=jnp.float32)
           + bl[...]
           + lax.dot_general(xv, wr[...], dn, preferred_element_type=jnp.float32)
           + br[...])
    nrm = jnp.maximum(jnp.sqrt(jnp.sum(out * out, axis=-1, keepdims=True)), 1e-12)
    out = out / nrm
    mu = jnp.mean(out, axis=-1, keepdims=True)
    var = jnp.mean((out - mu) ** 2, axis=-1, keepdims=True)
    out = (out - mu) * lax.rsqrt(var + 1e-5) * g[...] + beta[...]
    if not last:
      out = jnp.maximum(out, 0.0) + xv
    o_r[...] = out

  return pl.pallas_call(
      body,
      grid=(_NPAD // R,),
      in_specs=[
          pl.BlockSpec((R, _D), lambda i: (i, 0)),
          pl.BlockSpec((R, _D), lambda i: (i, 0)),
          pl.BlockSpec((R, 16), lambda i: (i, 0)),
          pl.BlockSpec((R, 16), lambda i: (i, 0)),
          pl.BlockSpec((R, _D), lambda i: (i, 0)),
          pl.BlockSpec((_D, _D), lambda i: (0, 0)),
          pl.BlockSpec((1, _D), lambda i: (0, 0)),
          pl.BlockSpec((_D, _D), lambda i: (0, 0)),
          pl.BlockSpec((1, _D), lambda i: (0, 0)),
          pl.BlockSpec((1, _D), lambda i: (0, 0)),
          pl.BlockSpec((1, _D), lambda i: (0, 0)),
      ],
      out_specs=pl.BlockSpec((R, _D), lambda i: (i, 0)),
      out_shape=jax.ShapeDtypeStruct((_NPAD, _D), jnp.float32),
  )


_dense_mid = _make_dense(False)
_dense_last = _make_dense(True)


def kernel(x, edge_index, Wl0, bl0, Wr0, br0, g0, beta0,
           Wl1, bl1, Wr1, br1, g1, beta1):
  src = edge_index[0]
  dst = edge_index[1]
  pad = _EPAD - _E
  ar = jnp.arange(pad, dtype=jnp.int32)
  srcp = jnp.concatenate([src, ar % _N]).reshape(_NW * _NCH, _K)
  dstp = jnp.concatenate(
      [dst, _N + (ar % (_NPAD - _N))]).reshape(_NW * _NCH, _K)
  xp = jnp.pad(x, ((0, _NPAD - _N), (0, 0)))
  xaug = jnp.concatenate([xp, jnp.ones((_NPAD, 16), jnp.float32)], axis=1)

  sd_flat = _sc_agg_deg(xaug, srcp, dstp, jnp.zeros((_RPT, _DW), jnp.float32))
  s0, s1 = sd_flat[:_NPAD, :_D], sd_flat[_NPAD:, :_D]
  d0, d1 = sd_flat[:_NPAD, _D:], sd_flat[_NPAD:, _D:]

  def v(a):
    return a.reshape(1, _D)

  h = _dense_mid(s0, s1, d0, d1, xp, Wl0, v(bl0), Wr0, v(br0), v(g0), v(beta0))

  s_flat2 = _sc_agg_nodeg(h, srcp, dstp,
                          jnp.zeros((_RPT, _D), jnp.float32))
  s0b, s1b = s_flat2[:_NPAD], s_flat2[_NPAD:]

  out = _dense_last(s0b, s1b, d0, d1, h, Wl1, v(bl1), Wr1, v(br1), v(g1),
                    v(beta1))
  return out[:_N]


# speedup vs baseline: 9.1491x; 1.5149x over previous
import functools

import jax
import jax.numpy as jnp
from jax import lax
from jax.experimental import pallas as pl
from jax.experimental.pallas import tpu as pltpu
from jax.experimental.pallas import tpu_sc as plsc

_N = 10000
_D = 128
_E = 320000

_NC = 2
_NS = 16
_NW = _NC * _NS

_K = 128
_NCH = 80
_NCHP = _NCH + 1
_EPAD = _NW * _NCHP * _K
_NPAD = 10048
_RPT = _NPAD // _NS


def _make_sc_agg(width):
  mesh = plsc.VectorSubcoreMesh(core_axis_name="c", subcore_axis_name="s")
  out_type = jax.ShapeDtypeStruct((_NC * _NPAD, width), jnp.float32)
  scratch = [
      pltpu.VMEM((_K,), jnp.int32),
      pltpu.VMEM((_K,), jnp.int32),
      pltpu.VMEM((_K,), jnp.int32),
      pltpu.VMEM((_K,), jnp.int32),
      pltpu.VMEM((_K, width), jnp.float32),
      pltpu.VMEM((_K, width), jnp.float32),
      pltpu.VMEM_SHARED((_NPAD, width), jnp.float32),
      pltpu.SemaphoreType.DMA,
      pltpu.SemaphoreType.DMA,
  ]

  def body(x_hbm, src_hbm, dst_hbm, z_hbm, out_sum,
           src_a, dst_a, src_b, dst_b, msgs_a, msgs_b, sum_sh, sem_a, sem_b):
    c = lax.axis_index("c")
    s = lax.axis_index("s")
    wid = s * _NC + c
    ebase = wid * _NCHP
    base = s * _RPT

    pltpu.sync_copy(z_hbm, sum_sh.at[pl.ds(base, _RPT)])
    plsc.subcore_barrier()

    pltpu.sync_copy(src_hbm.at[ebase], src_a)
    pltpu.sync_copy(dst_hbm.at[ebase], dst_a)
    pltpu.async_copy(x_hbm.at[src_a], msgs_a, sem_a)

    def outer(jj, carry):
      j0 = 2 * jj
      pltpu.sync_copy(src_hbm.at[ebase + j0 + 1], src_b)
      pltpu.sync_copy(dst_hbm.at[ebase + j0 + 1], dst_b)
      pltpu.async_copy(x_hbm.at[src_b], msgs_b, sem_b)
      pltpu.make_async_copy(x_hbm.at[src_a], msgs_a, sem_a).wait()
      pltpu.sync_copy(msgs_a, sum_sh.at[dst_a], add=True)
      pltpu.sync_copy(src_hbm.at[ebase + j0 + 2], src_a)
      pltpu.sync_copy(dst_hbm.at[ebase + j0 + 2], dst_a)
      pltpu.async_copy(x_hbm.at[src_a], msgs_a, sem_a)
      pltpu.make_async_copy(x_hbm.at[src_b], msgs_b, sem_b).wait()
      pltpu.sync_copy(msgs_b, sum_sh.at[dst_b], add=True)
      return carry

    lax.fori_loop(0, _NCH // 2, outer, 0)
    pltpu.make_async_copy(x_hbm.at[src_a], msgs_a, sem_a).wait()
    plsc.subcore_barrier()

    obase = c * _NPAD + base
    pltpu.sync_copy(sum_sh.at[pl.ds(base, _RPT)], out_sum.at[pl.ds(obase, _RPT)])

  return pl.kernel(
      body, out_type=out_type, mesh=mesh, scratch_types=scratch,
      compiler_params=pltpu.CompilerParams(use_tc_tiling_on_sc=False))


_DW = _D + 16
_sc_agg_deg = _make_sc_agg(_DW)
_sc_agg_nodeg = _make_sc_agg(_D)


_R = 1256
_NB = _NPAD // _R


def _make_dense(last):
  sw = _DW if not last else _D

  def body(sd0, sd1, g0_r, g1_r, x_r, wl, bl, wr, br, g, beta, o_r):
    ssum = sd0[:, :_D] + sd1[:, :_D]
    deg = g0_r[:, _D:_D + 1] + g1_r[:, _D:_D + 1]
    degc = jnp.maximum(deg, 1.0)
    mean = ssum / degc
    xv = x_r[:, :_D]
    dn = (((1,), (1,)), ((), ()))
    out = (lax.dot_general(mean, wl[...], dn, preferred_element_type=jnp.float32)
           + bl[...]
           + lax.dot_general(xv, wr[...], dn, preferred_element_type=jnp.float32)
           + br[...])
    nrm = jnp.maximum(jnp.sqrt(jnp.sum(out * out, axis=-1, keepdims=True)), 1e-12)
    out = out / nrm
    mu = jnp.mean(out, axis=-1, keepdims=True)
    var = jnp.mean((out - mu) ** 2, axis=-1, keepdims=True)
    out = (out - mu) * lax.rsqrt(var + 1e-5) * g[...] + beta[...]
    if not last:
      out = jnp.maximum(out, 0.0) + xv
    o_r[...] = out

  xw = _DW if not last else _D
  return pl.pallas_call(
      body,
      grid=(_NB,),
      in_specs=[
          pl.BlockSpec((_R, sw), lambda i: (i, 0)),
          pl.BlockSpec((_R, sw), lambda i: (i + _NB, 0)),
          pl.BlockSpec((_R, _DW), lambda i: (i, 0)),
          pl.BlockSpec((_R, _DW), lambda i: (i + _NB, 0)),
          pl.BlockSpec((_R, xw), lambda i: (i, 0)),
          pl.BlockSpec((_D, _D), lambda i: (0, 0)),
          pl.BlockSpec((1, _D), lambda i: (0, 0)),
          pl.BlockSpec((_D, _D), lambda i: (0, 0)),
          pl.BlockSpec((1, _D), lambda i: (0, 0)),
          pl.BlockSpec((1, _D), lambda i: (0, 0)),
          pl.BlockSpec((1, _D), lambda i: (0, 0)),
      ],
      out_specs=pl.BlockSpec((_R, _D), lambda i: (i, 0)),
      out_shape=jax.ShapeDtypeStruct((_NPAD, _D), jnp.float32),
  )


_dense_mid = _make_dense(False)
_dense_last = _make_dense(True)


def kernel(x, edge_index, Wl0, bl0, Wr0, br0, g0, beta0,
           Wl1, bl1, Wr1, br1, g1, beta1):
  src = edge_index[0]
  dst = edge_index[1]
  pad = _NW * _NCH * _K - _E
  ar = jnp.arange(pad, dtype=jnp.int32)
  ar2 = jnp.arange(_NW * _K, dtype=jnp.int32)
  srcp = jnp.concatenate([
      jnp.concatenate([src, ar % _N]).reshape(_NW, _NCH, _K),
      (ar2 % _N).reshape(_NW, 1, _K)], axis=1).reshape(_NW * _NCHP, _K)
  dstp = jnp.concatenate([
      jnp.concatenate([dst, _N + (ar % (_NPAD - _N))]).reshape(_NW, _NCH, _K),
      jnp.full((_NW, 1, _K), _N, jnp.int32)], axis=1).reshape(_NW * _NCHP, _K)
  xp = jnp.pad(x, ((0, _NPAD - _N), (0, 0)))
  xaug = jnp.concatenate([xp, jnp.ones((_NPAD, 16), jnp.float32)], axis=1)

  sd = _sc_agg_deg(xaug, srcp, dstp, jnp.zeros((_RPT, _DW), jnp.float32))

  def v(a):
    return a.reshape(1, _D)

  h = _dense_mid(sd, sd, sd, sd, xaug,
                 Wl0, v(bl0), Wr0, v(br0), v(g0), v(beta0))

  s2 = _sc_agg_nodeg(h, srcp, dstp, jnp.zeros((_RPT, _D), jnp.float32))

  out = _dense_last(s2, s2, sd, sd, h,
                    Wl1, v(bl1), Wr1, v(br1), v(g1), v(beta1))
  return out[:_N]


# speedup vs baseline: 10.0652x; 1.1001x over previous
import functools

import jax
import jax.numpy as jnp
from jax import lax
from jax.experimental import pallas as pl
from jax.experimental.pallas import tpu as pltpu
from jax.experimental.pallas import tpu_sc as plsc

_N = 10000
_D = 128
_E = 320000

_NC = 2
_NS = 16
_NW = _NC * _NS

_K = 112
_NCH = 92
_G = 4
_NG = _NCH // _G
_EPAD = _NW * _NCH * _K
_NPAD = 10048
_RPT = _NPAD // _NS


def _make_sc_agg(width):
  mesh = plsc.VectorSubcoreMesh(core_axis_name="c", subcore_axis_name="s")
  out_type = jax.ShapeDtypeStruct((_NC * _NPAD, width), jnp.float32)
  scratch = [
      pltpu.VMEM((2 * _G, _K), jnp.int32),
      pltpu.VMEM((2 * _G, _K), jnp.int32),
      pltpu.VMEM((_K, width), jnp.float32),
      pltpu.VMEM((_K, width), jnp.float32),
      pltpu.VMEM_SHARED((_NPAD, width), jnp.float32),
      pltpu.SemaphoreType.DMA,
      pltpu.SemaphoreType.DMA,
  ]

  def body(x_hbm, e_hbm, z_hbm, out_sum,
           gbuf_a, gbuf_b, msgs_a, msgs_b, sum_sh, sem_a, sem_b):
    c = lax.axis_index("c")
    s = lax.axis_index("s")
    wid = s * _NC + c
    ebase = wid * (2 * _NCH)
    base = s * _RPT
    gbuf = (gbuf_a, gbuf_b)
    msgs = (msgs_a, msgs_b)
    sems = (sem_a, sem_b)

    pltpu.sync_copy(z_hbm, sum_sh.at[pl.ds(base, _RPT)])
    plsc.subcore_barrier()

    pltpu.sync_copy(e_hbm.at[pl.ds(ebase, 2 * _G)], gbuf_a)
    pltpu.async_copy(x_hbm.at[gbuf_a.at[0]], msgs_a, sems[0])

    for n in range(_NG):
      gcur = gbuf[n % 2]
      gnext = gbuf[(n + 1) % 2]
      if n + 1 < _NG:
        pltpu.sync_copy(
            e_hbm.at[pl.ds(ebase + (n + 1) * 2 * _G, 2 * _G)], gnext)
      for g in range(_G):
        j = n * _G + g
        if j + 1 < _NCH:
          nsrc = gcur.at[2 * (g + 1)] if g < _G - 1 else gnext.at[0]
          pltpu.async_copy(x_hbm.at[nsrc], msgs[(j + 1) % 2],
                           sems[(j + 1) % 2])
        pltpu.make_async_copy(
            x_hbm.at[gcur.at[2 * g]], msgs[j % 2], sems[j % 2]).wait()
        pltpu.sync_copy(msgs[j % 2], sum_sh.at[gcur.at[2 * g + 1]], add=True)

    plsc.subcore_barrier()
    obase = c * _NPAD + base
    pltpu.sync_copy(sum_sh.at[pl.ds(base, _RPT)], out_sum.at[pl.ds(obase, _RPT)])

  return pl.kernel(
      body, out_type=out_type, mesh=mesh, scratch_types=scratch,
      compiler_params=pltpu.CompilerParams(use_tc_tiling_on_sc=False))


_DW = _D + 16
_sc_agg_deg = _make_sc_agg(_DW)
_sc_agg_nodeg = _make_sc_agg(_D)


_R = 1256
_NB = _NPAD // _R


def _make_dense(last):
  sw = _DW if not last else _D

  def body(sd0, sd1, g0_r, g1_r, x_r, wl, bl, wr, br, g, beta, o_r):
    ssum = sd0[:, :_D] + sd1[:, :_D]
    deg = g0_r[:, _D:_D + 1] + g1_r[:, _D:_D + 1]
    degc = jnp.maximum(deg, 1.0)
    mean = ssum / degc
    xv = x_r[:, :_D]
    dn = (((1,), (1,)), ((), ()))
    out = (lax.dot_general(mean, wl[...], dn, preferred_element_type=jnp.float32)
           + bl[...]
           + lax.dot_general(xv, wr[...], dn, preferred_element_type=jnp.float32)
           + br[...])
    nrm = jnp.maximum(jnp.sqrt(jnp.sum(out * out, axis=-1, keepdims=True)), 1e-12)
    out = out / nrm
    mu = jnp.mean(out, axis=-1, keepdims=True)
    var = jnp.mean((out - mu) ** 2, axis=-1, keepdims=True)
    out = (out - mu) * lax.rsqrt(var + 1e-5) * g[...] + beta[...]
    if not last:
      out = jnp.maximum(out, 0.0) + xv
    o_r[...] = out

  xw = _DW if not last else _D
  return pl.pallas_call(
      body,
      grid=(_NB,),
      in_specs=[
          pl.BlockSpec((_R, sw), lambda i: (i, 0)),
          pl.BlockSpec((_R, sw), lambda i: (i + _NB, 0)),
          pl.BlockSpec((_R, _DW), lambda i: (i, 0)),
          pl.BlockSpec((_R, _DW), lambda i: (i + _NB, 0)),
          pl.BlockSpec((_R, xw), lambda i: (i, 0)),
          pl.BlockSpec((_D, _D), lambda i: (0, 0)),
          pl.BlockSpec((1, _D), lambda i: (0, 0)),
          pl.BlockSpec((_D, _D), lambda i: (0, 0)),
          pl.BlockSpec((1, _D), lambda i: (0, 0)),
          pl.BlockSpec((1, _D), lambda i: (0, 0)),
          pl.BlockSpec((1, _D), lambda i: (0, 0)),
      ],
      out_specs=pl.BlockSpec((_R, _D), lambda i: (i, 0)),
      out_shape=jax.ShapeDtypeStruct((_NPAD, _D), jnp.float32),
  )


_dense_mid = _make_dense(False)
_dense_last = _make_dense(True)


def kernel(x, edge_index, Wl0, bl0, Wr0, br0, g0, beta0,
           Wl1, bl1, Wr1, br1, g1, beta1):
  src = edge_index[0]
  dst = edge_index[1]
  pad = _EPAD - _E
  ar = jnp.arange(pad, dtype=jnp.int32)
  srcp = jnp.concatenate([src, ar % _N]).reshape(_NW, _NCH, 1, _K)
  dstp = jnp.concatenate(
      [dst, _N + (ar % (_NPAD - _N))]).reshape(_NW, _NCH, 1, _K)
  edges = jnp.concatenate([srcp, dstp], axis=2).reshape(_NW * _NCH * 2, _K)
  xp = jnp.pad(x, ((0, _NPAD - _N), (0, 0)))
  xaug = jnp.concatenate([xp, jnp.ones((_NPAD, 16), jnp.float32)], axis=1)

  sd = _sc_agg_deg(xaug, edges, jnp.zeros((_RPT, _DW), jnp.float32))

  def v(a):
    return a.reshape(1, _D)

  h = _dense_mid(sd, sd, sd, sd, xaug,
                 Wl0, v(bl0), Wr0, v(br0), v(g0), v(beta0))

  s2 = _sc_agg_nodeg(h, edges, jnp.zeros((_RPT, _D), jnp.float32))

  out = _dense_last(s2, s2, sd, sd, h,
                    Wl1, v(bl1), Wr1, v(br1), v(g1), v(beta1))
  return out[:_N]


# speedup vs baseline: 10.2324x; 1.0166x over previous
import functools

import jax
import jax.numpy as jnp
from jax import lax
from jax.experimental import pallas as pl
from jax.experimental.pallas import tpu as pltpu
from jax.experimental.pallas import tpu_sc as plsc

_N = 10000
_D = 128
_E = 320000

_NC = 2
_NS = 16
_NW = _NC * _NS

_K = 80
_NCH = 128
_G = 4
_NG = _NCH // _G
_EPAD = _NW * _NCH * _K
_NPAD = 10048
_RPT = _NPAD // _NS


def _make_sc_agg(width):
  mesh = plsc.VectorSubcoreMesh(core_axis_name="c", subcore_axis_name="s")
  out_type = jax.ShapeDtypeStruct((_NC * _NPAD, width), jnp.float32)
  scratch = [
      pltpu.VMEM((2 * _G, _K), jnp.int32),
      pltpu.VMEM((2 * _G, _K), jnp.int32),
      pltpu.VMEM((_K, width), jnp.float32),
      pltpu.VMEM((_K, width), jnp.float32),
      pltpu.VMEM((_K, width), jnp.float32),
      pltpu.VMEM_SHARED((_NPAD, width), jnp.float32),
      pltpu.SemaphoreType.DMA,
      pltpu.SemaphoreType.DMA,
      pltpu.SemaphoreType.DMA,
      pltpu.SemaphoreType.DMA,
      pltpu.SemaphoreType.DMA,
      pltpu.SemaphoreType.DMA,
  ]

  def body(x_hbm, e_hbm, z_hbm, out_sum,
           gbuf_a, gbuf_b, msgs_0, msgs_1, msgs_2, sum_sh,
           sg0, sg1, sg2, ss0, ss1, ss2):
    c = lax.axis_index("c")
    s = lax.axis_index("s")
    wid = s * _NC + c
    ebase = wid * (2 * _NCH)
    base = s * _RPT
    gbuf = (gbuf_a, gbuf_b)
    msgs = (msgs_0, msgs_1, msgs_2)
    sem_g = (sg0, sg1, sg2)
    sem_s = (ss0, ss1, ss2)

    pltpu.sync_copy(z_hbm, sum_sh.at[pl.ds(base, _RPT)])
    plsc.subcore_barrier()

    def srcrow(j):
      return gbuf[(j // _G) % 2].at[2 * (j % _G)]

    def dstrow(j):
      return gbuf[(j // _G) % 2].at[2 * (j % _G) + 1]

    pltpu.sync_copy(e_hbm.at[pl.ds(ebase, 2 * _G)], gbuf_a)
    pltpu.async_copy(x_hbm.at[srcrow(0)], msgs[0], sem_g[0])

    for n in range(_NG):
      gnext = gbuf[(n + 1) % 2]
      for g in range(_G):
        j = n * _G + g
        b = j % 3
        if g == 2 and n + 1 < _NG:
          pltpu.sync_copy(
              e_hbm.at[pl.ds(ebase + (n + 1) * 2 * _G, 2 * _G)], gnext)
        if j + 1 < _NCH:
          bn = (j + 1) % 3
          if j >= 2:
            pltpu.make_async_copy(
                msgs[bn], sum_sh.at[dstrow(j - 2)], sem_s[bn]).wait()
          pltpu.async_copy(x_hbm.at[srcrow(j + 1)], msgs[bn], sem_g[bn])
        pltpu.make_async_copy(
            x_hbm.at[srcrow(j)], msgs[b], sem_g[b]).wait()
        pltpu.async_copy(msgs[b], sum_sh.at[dstrow(j)], sem_s[b], add=True)

    for j in (_NCH - 3, _NCH - 2, _NCH - 1):
      pltpu.make_async_copy(
          msgs[j % 3], sum_sh.at[dstrow(j)], sem_s[j % 3]).wait()
    plsc.subcore_barrier()
    obase = c * _NPAD + base
    pltpu.sync_copy(sum_sh.at[pl.ds(base, _RPT)], out_sum.at[pl.ds(obase, _RPT)])

  return pl.kernel(
      body, out_type=out_type, mesh=mesh, scratch_types=scratch,
      compiler_params=pltpu.CompilerParams(use_tc_tiling_on_sc=False))


_DW = _D + 16
_sc_agg_deg = _make_sc_agg(_DW)
_sc_agg_nodeg = _make_sc_agg(_D)


_R = 1256
_NB = _NPAD // _R


def _make_dense(last):
  sw = _DW if not last else _D

  def body(sd0, sd1, g0_r, g1_r, x_r, wl, bl, wr, br, g, beta, o_r):
    ssum = sd0[:, :_D] + sd1[:, :_D]
    deg = g0_r[:, _D:_D + 1] + g1_r[:, _D:_D + 1]
    degc = jnp.maximum(deg, 1.0)
    mean = ssum / degc
    xv = x_r[:, :_D]
    dn = (((1,), (1,)), ((), ()))
    out = (lax.dot_general(mean, wl[...], dn, preferred_element_type=jnp.float32)
           + bl[...]
           + lax.dot_general(xv, wr[...], dn, preferred_element_type=jnp.float32)
           + br[...])
    nrm = jnp.maximum(jnp.sqrt(jnp.sum(out * out, axis=-1, keepdims=True)), 1e-12)
    out = out / nrm
    mu = jnp.mean(out, axis=-1, keepdims=True)
    var = jnp.mean((out - mu) ** 2, axis=-1, keepdims=True)
    out = (out - mu) * lax.rsqrt(var + 1e-5) * g[...] + beta[...]
    if not last:
      out = jnp.maximum(out, 0.0) + xv
    o_r[...] = out

  xw = _DW if not last else _D
  return pl.pallas_call(
      body,
      grid=(_NB,),
      in_specs=[
          pl.BlockSpec((_R, sw), lambda i: (i, 0)),
          pl.BlockSpec((_R, sw), lambda i: (i + _NB, 0)),
          pl.BlockSpec((_R, _DW), lambda i: (i, 0)),
          pl.BlockSpec((_R, _DW), lambda i: (i + _NB, 0)),
          pl.BlockSpec((_R, xw), lambda i: (i, 0)),
          pl.BlockSpec((_D, _D), lambda i: (0, 0)),
          pl.BlockSpec((1, _D), lambda i: (0, 0)),
          pl.BlockSpec((_D, _D), lambda i: (0, 0)),
          pl.BlockSpec((1, _D), lambda i: (0, 0)),
          pl.BlockSpec((1, _D), lambda i: (0, 0)),
          pl.BlockSpec((1, _D), lambda i: (0, 0)),
      ],
      out_specs=pl.BlockSpec((_R, _D), lambda i: (i, 0)),
      out_shape=jax.ShapeDtypeStruct((_NPAD, _D), jnp.float32),
  )


_dense_mid = _make_dense(False)
_dense_last = _make_dense(True)


def kernel(x, edge_index, Wl0, bl0, Wr0, br0, g0, beta0,
           Wl1, bl1, Wr1, br1, g1, beta1):
  src = edge_index[0]
  dst = edge_index[1]
  pad = _EPAD - _E
  ar = jnp.arange(pad, dtype=jnp.int32)
  srcp = jnp.concatenate([src, ar % _N]).reshape(_NW, _NCH, 1, _K)
  dstp = jnp.concatenate(
      [dst, _N + (ar % (_NPAD - _N))]).reshape(_NW, _NCH, 1, _K)
  edges = jnp.concatenate([srcp, dstp], axis=2).reshape(_NW * _NCH * 2, _K)
  xp = jnp.pad(x, ((0, _NPAD - _N), (0, 0)))
  xaug = jnp.concatenate([xp, jnp.ones((_NPAD, 16), jnp.float32)], axis=1)

  sd = _sc_agg_deg(xaug, edges, jnp.zeros((_RPT, _DW), jnp.float32))

  def v(a):
    return a.reshape(1, _D)

  h = _dense_mid(sd, sd, sd, sd, xaug,
                 Wl0, v(bl0), Wr0, v(br0), v(g0), v(beta0))

  s2 = _sc_agg_nodeg(h, edges, jnp.zeros((_RPT, _D), jnp.float32))

  out = _dense_last(s2, s2, sd, sd, h,
                    Wl1, v(bl1), Wr1, v(br1), v(g1), v(beta1))
  return out[:_N]
